# baseline (speedup 1.0000x reference)
import functools

import jax
import jax.numpy as jnp
from jax import lax
from jax.experimental import pallas as pl
from jax.experimental.pallas import tpu as pltpu
from jax.experimental.pallas import tpu_sc as plsc

_N = 10000
_E = 320000
_D = 128
_NW = 32
_NLOC = 320
_RPW = 328
_NPAD_BH = 10248
_NPAD_C = 10240
_B = 1280
_NBLK = _E // _B
_K = 64
_G = _D // 16


def _mm_body(x_ref, wa_ref, ba_ref, wb_ref, bb_ref, ab_ref, bh_ref):
    x = x_ref[...]
    ah = jnp.dot(x, wa_ref[...], preferred_element_type=jnp.float32) + ba_ref[...]
    bh = jnp.dot(x, wb_ref[...], preferred_element_type=jnp.float32) + bb_ref[...]
    ab_ref[:, : _D] = ah
    ab_ref[:, _D :] = bh
    bh_ref[...] = bh


def _mm_phase(x, W_A, b_A, W_B, b_B):
    bm = 2000
    grid = (_N // bm,)
    return pl.pallas_call(
        _mm_body,
        grid=grid,
        in_specs=[
            pl.BlockSpec((bm, _D), lambda i: (i, 0)),
            pl.BlockSpec((_D, _D), lambda i: (0, 0)),
            pl.BlockSpec((1, _D), lambda i: (0, 0)),
            pl.BlockSpec((_D, _D), lambda i: (0, 0)),
            pl.BlockSpec((1, _D), lambda i: (0, 0)),
        ],
        out_specs=[
            pl.BlockSpec((bm, 2 * _D), lambda i: (i, 0)),
            pl.BlockSpec((bm, _D), lambda i: (i, 0)),
        ],
        out_shape=[
            jax.ShapeDtypeStruct((_N, 2 * _D), jnp.float32),
            jax.ShapeDtypeStruct((_NPAD_BH, _D), jnp.float32),
        ],
    )(x, W_A, b_A, W_B, b_B)


def _edge_body(ab_hbm, bh_hbm, src_hbm, dst_hbm, c_hbm,
               acc, bh_loc, src_blk, dst_blk, q_src, q_loc, rows,
               sem_i0, sem_i1, sem_g0, sem_g1):
    wid = lax.axis_index("s") * 2 + lax.axis_index("c")
    lo = wid * _NLOC

    zero_f = jnp.zeros((16,), jnp.float32)

    def _zero_row(r, carry):
        for g in range(_G):
            acc[r, pl.ds(g * 16, 16)] = zero_f
        return carry

    lax.fori_loop(0, _RPW, _zero_row, 0)

    pltpu.sync_copy(bh_hbm.at[pl.ds(lo, _RPW)], bh_loc)

    zero_i = jnp.zeros((16,), jnp.int32)
    trash_i = jnp.full((16,), _NLOC, jnp.int32)

    _QW = _B + _K

    def _start_idx(par, sem, b):
        off = jnp.minimum(b, _NBLK - 1) * _B
        pltpu.async_copy(
            src_hbm.at[pl.ds(off, _B)], src_blk.at[pl.ds(par * _B, _B)], sem
        )
        pltpu.async_copy(
            dst_hbm.at[pl.ds(off, _B)], dst_blk.at[pl.ds(par * _B, _B)], sem
        )

    def _wait_idx(par, sem):
        pltpu.make_async_copy(
            src_hbm.at[pl.ds(0, _B)], src_blk.at[pl.ds(par * _B, _B)], sem
        ).wait()
        pltpu.make_async_copy(
            dst_hbm.at[pl.ds(0, _B)], dst_blk.at[pl.ds(par * _B, _B)], sem
        ).wait()

    def _filter(par):
        def _filt(i, qn):
            s = src_blk[pl.ds(par * _B + i * 16, 16)]
            d = dst_blk[pl.ds(par * _B + i * 16, 16)]
            locv = d - lo
            mask = (locv >= 0) & (locv < _NLOC)
            csum = plsc.cumsum(mask.astype(jnp.int32))
            pos = csum + (par * _QW + qn - 1)
            plsc.store_scatter(q_src, [pos], s, mask=mask)
            plsc.store_scatter(q_loc, [pos], locv, mask=mask)
            return qn + csum[15]

        qn = lax.fori_loop(0, _B // 16, _filt, jnp.int32(0))
        for t in range(_K // 16):
            q_src[pl.ds(par * _QW + qn + t * 16, 16)] = zero_i
            q_loc[pl.ds(par * _QW + qn + t * 16, 16)] = trash_i
        return qn

    def _start_gather(par, sem, cbase):
        return pltpu.async_copy(
            ab_hbm.at[q_src.at[pl.ds(par * _QW + cbase, _K)]],
            rows.at[pl.ds(par * _K, _K)],
            sem,
        )

    def _grp(par, cbase, g16, carry):
        qbase = par * _QW + cbase + g16 * 16
        locv = q_loc[pl.ds(qbase, 16)]
        for j in range(16):
            e = par * _K + g16 * 16 + j
            locj = locv[j]
            for g in range(_G):
                a = rows[e, pl.ds(g * 16, 16)]
                bs = rows[e, pl.ds(_D + g * 16, 16)]
                bd = bh_loc[locj, pl.ds(g * 16, 16)]
                sig = 1.0 / (1.0 + jnp.exp(-(bs + bd)))
                m = jnp.maximum(sig * a, 0.0)
                cur = acc[locj, pl.ds(g * 16, 16)]
                acc[locj, pl.ds(g * 16, 16)] = jnp.maximum(cur, m)
        return carry

    def _process(par, sem, qn, gdesc):
        gdesc.wait()
        ngrp_total = (qn + 15) // 16
        nchunks = (qn + _K - 1) // _K

        def _chunk(ch, carry):
            cbase = ch * _K

            @pl.when(ch > 0)
            def _():
                _start_gather(par, sem, cbase).wait()

            ngrp = jnp.minimum(ngrp_total - ch * (_K // 16), _K // 16)
            lax.fori_loop(0, ngrp, lambda g16, c: _grp(par, cbase, g16, c), 0)
            return carry

        lax.fori_loop(0, nchunks, _chunk, 0)

    _start_idx(0, sem_i0, jnp.int32(0))
    _start_idx(1, sem_i1, jnp.int32(1))

    def _pair(p, carry):
        b0 = 2 * p
        _wait_idx(0, sem_i0)
        qn0 = _filter(0)
        _start_idx(0, sem_i0, b0 + 2)
        g0 = _start_gather(0, sem_g0, 0)

        _wait_idx(1, sem_i1)
        qn1 = _filter(1)
        _start_idx(1, sem_i1, b0 + 3)
        g1 = _start_gather(1, sem_g1, 0)

        _process(0, sem_g0, qn0, g0)
        _process(1, sem_g1, qn1, g1)
        return carry

    lax.fori_loop(0, _NBLK // 2, _pair, 0)

    _wait_idx(0, sem_i0)
    _wait_idx(1, sem_i1)

    pltpu.sync_copy(acc.at[pl.ds(0, _NLOC)], c_hbm.at[pl.ds(lo, _NLOC)])


def _edge_phase(ab, bh, src, dst):
    mesh = plsc.VectorSubcoreMesh(core_axis_name="c", subcore_axis_name="s")
    kern = functools.partial(
        pl.kernel,
        out_type=jax.ShapeDtypeStruct((_NPAD_C, _D), jnp.float32),
        mesh=mesh,
        scratch_types=[
            pltpu.VMEM((_RPW, _D), jnp.float32),
            pltpu.VMEM((_RPW, _D), jnp.float32),
            pltpu.VMEM((2 * _B,), jnp.int32),
            pltpu.VMEM((2 * _B,), jnp.int32),
            pltpu.VMEM((2 * (_B + _K),), jnp.int32),
            pltpu.VMEM((2 * (_B + _K),), jnp.int32),
            pltpu.VMEM((2 * _K, 2 * _D), jnp.float32),
            pltpu.SemaphoreType.DMA,
            pltpu.SemaphoreType.DMA,
            pltpu.SemaphoreType.DMA,
            pltpu.SemaphoreType.DMA,
        ],
        compiler_params=pltpu.CompilerParams(needs_layout_passes=False),
    )(_edge_body)
    return kern(ab, bh, src, dst)


def _apply_body(x_ref, c_ref, w_ref, bap_ref, o_ref):
    h = x_ref[...]
    c = c_ref[...]
    b = (
        jnp.dot(h, w_ref[: _D, :], preferred_element_type=jnp.float32)
        + jnp.dot(c, w_ref[_D :, :], preferred_element_type=jnp.float32)
        + bap_ref[...]
    )
    nrm = jnp.sqrt(jnp.sum(b * b, axis=1, keepdims=True))
    b = b / jnp.maximum(nrm, 1e-12)
    o_ref[...] = jnp.maximum(b, 0.0)


def _apply_phase(x, c_pad, W_apply, b_apply):
    bm = 2000
    grid = (_N // bm,)
    return pl.pallas_call(
        _apply_body,
        grid=grid,
        in_specs=[
            pl.BlockSpec((bm, _D), lambda i: (i, 0)),
            pl.BlockSpec((bm, _D), lambda i: (i, 0)),
            pl.BlockSpec((2 * _D, _D), lambda i: (0, 0)),
            pl.BlockSpec((1, _D), lambda i: (0, 0)),
        ],
        out_specs=pl.BlockSpec((bm, _D), lambda i: (i, 0)),
        out_shape=jax.ShapeDtypeStruct((_N, _D), jnp.float32),
    )(x, c_pad, W_apply, b_apply)


def kernel(x, edge_index, W_A, b_A, W_B, b_B, W_apply, b_apply):
    ab, bh = _mm_phase(
        x, W_A, b_A.reshape(1, _D), W_B, b_B.reshape(1, _D)
    )
    c_pad = _edge_phase(ab, bh, edge_index[0], edge_index[1])
    return _apply_phase(x, c_pad, W_apply, b_apply.reshape(1, _D))


# speedup vs baseline: 1.0741x; 1.0741x over previous
import functools

import jax
import jax.numpy as jnp
from jax import lax
from jax.experimental import pallas as pl
from jax.experimental.pallas import tpu as pltpu
from jax.experimental.pallas import tpu_sc as plsc

_N = 10000
_E = 320000
_D = 128
_NW = 32
_NLOC = 320
_RPW = 328
_NPAD_BH = 10248
_NPAD_C = 10240
_B = 1280
_NBLK = _E // _B
_K = 64
_G = _D // 16


def _mm_body(x_ref, wa_ref, ba_ref, wb_ref, bb_ref, ab_ref, bh_ref):
    x = x_ref[...]
    ah = jnp.dot(x, wa_ref[...], preferred_element_type=jnp.float32) + ba_ref[...]
    bh = jnp.dot(x, wb_ref[...], preferred_element_type=jnp.float32) + bb_ref[...]
    ab_ref[:, : _D] = jnp.maximum(ah, 0.0)
    ab_ref[:, _D :] = bh
    bh_ref[...] = bh


def _mm_phase(x, W_A, b_A, W_B, b_B):
    bm = 2000
    grid = (_N // bm,)
    return pl.pallas_call(
        _mm_body,
        grid=grid,
        in_specs=[
            pl.BlockSpec((bm, _D), lambda i: (i, 0)),
            pl.BlockSpec((_D, _D), lambda i: (0, 0)),
            pl.BlockSpec((1, _D), lambda i: (0, 0)),
            pl.BlockSpec((_D, _D), lambda i: (0, 0)),
            pl.BlockSpec((1, _D), lambda i: (0, 0)),
        ],
        out_specs=[
            pl.BlockSpec((bm, 2 * _D), lambda i: (i, 0)),
            pl.BlockSpec((bm, _D), lambda i: (i, 0)),
        ],
        out_shape=[
            jax.ShapeDtypeStruct((_N, 2 * _D), jnp.float32),
            jax.ShapeDtypeStruct((_NPAD_BH, _D), jnp.float32),
        ],
    )(x, W_A, b_A, W_B, b_B)


def _edge_body(ab_hbm, bh_hbm, src_hbm, dst_hbm, c_hbm,
               acc, bh_loc, src_blk, dst_blk, q_src, q_loc, rows,
               sem_i0, sem_i1, sem_g0, sem_g1):
    wid = lax.axis_index("s") * 2 + lax.axis_index("c")
    lo = wid * _NLOC

    zero_f = jnp.zeros((16,), jnp.float32)

    def _zero_row(r, carry):
        for g in range(_G):
            acc[r, pl.ds(g * 16, 16)] = zero_f
        return carry

    lax.fori_loop(0, _RPW, _zero_row, 0)

    pltpu.sync_copy(bh_hbm.at[pl.ds(lo, _RPW)], bh_loc)

    zero_i = jnp.zeros((16,), jnp.int32)
    trash_i = jnp.full((16,), _NLOC, jnp.int32)

    _QW = _B + _K

    def _start_idx(par, sem, b):
        off = jnp.minimum(b, _NBLK - 1) * _B
        pltpu.async_copy(
            src_hbm.at[pl.ds(off, _B)], src_blk.at[pl.ds(par * _B, _B)], sem
        )
        pltpu.async_copy(
            dst_hbm.at[pl.ds(off, _B)], dst_blk.at[pl.ds(par * _B, _B)], sem
        )

    def _wait_idx(par, sem):
        pltpu.make_async_copy(
            src_hbm.at[pl.ds(0, _B)], src_blk.at[pl.ds(par * _B, _B)], sem
        ).wait()
        pltpu.make_async_copy(
            dst_hbm.at[pl.ds(0, _B)], dst_blk.at[pl.ds(par * _B, _B)], sem
        ).wait()

    def _filter(par):
        def _filt(i, qn):
            s = src_blk[pl.ds(par * _B + i * 16, 16)]
            d = dst_blk[pl.ds(par * _B + i * 16, 16)]
            locv = d - lo
            mask = (locv >= 0) & (locv < _NLOC)
            csum = plsc.cumsum(mask.astype(jnp.int32))
            pos = csum + (par * _QW + qn - 1)
            plsc.store_scatter(q_src, [pos], s, mask=mask)
            plsc.store_scatter(q_loc, [pos], locv, mask=mask)
            return qn + csum[15]

        qn = lax.fori_loop(0, _B // 16, _filt, jnp.int32(0))
        for t in range(_K // 16):
            q_src[pl.ds(par * _QW + qn + t * 16, 16)] = zero_i
            q_loc[pl.ds(par * _QW + qn + t * 16, 16)] = trash_i
        return qn

    def _start_gather(par, sem, cbase):
        return pltpu.async_copy(
            ab_hbm.at[q_src.at[pl.ds(par * _QW + cbase, _K)]],
            rows.at[pl.ds(par * _K, _K)],
            sem,
        )

    def _grp(par, cbase, g16, carry):
        qbase = par * _QW + cbase + g16 * 16
        locv = q_loc[pl.ds(qbase, 16)]
        for j in range(16):
            e = par * _K + g16 * 16 + j
            locj = locv[j]
            bs = [rows[e, pl.ds(_D + g * 16, 16)] for g in range(_G)]
            bd = [bh_loc[locj, pl.ds(g * 16, 16)] for g in range(_G)]
            ex = [jnp.exp(-(bs[g] + bd[g])) for g in range(_G)]
            den = [1.0 + ex[g] for g in range(_G)]
            sig = [1.0 / den[g] for g in range(_G)]
            ap = [rows[e, pl.ds(g * 16, 16)] for g in range(_G)]
            m = [sig[g] * ap[g] for g in range(_G)]
            cur = [acc[locj, pl.ds(g * 16, 16)] for g in range(_G)]
            for g in range(_G):
                acc[locj, pl.ds(g * 16, 16)] = jnp.maximum(cur[g], m[g])
        return carry

    def _process(par, sem, qn, gdesc):
        gdesc.wait()
        ngrp_total = (qn + 15) // 16
        nchunks = (qn + _K - 1) // _K

        def _chunk(ch, carry):
            cbase = ch * _K

            @pl.when(ch > 0)
            def _():
                _start_gather(par, sem, cbase).wait()

            ngrp = jnp.minimum(ngrp_total - ch * (_K // 16), _K // 16)
            lax.fori_loop(0, ngrp, lambda g16, c: _grp(par, cbase, g16, c), 0)
            return carry

        lax.fori_loop(0, nchunks, _chunk, 0)

    _start_idx(0, sem_i0, jnp.int32(0))
    _start_idx(1, sem_i1, jnp.int32(1))

    def _pair(p, carry):
        b0 = 2 * p
        _wait_idx(0, sem_i0)
        qn0 = _filter(0)
        _start_idx(0, sem_i0, b0 + 2)
        g0 = _start_gather(0, sem_g0, 0)

        _wait_idx(1, sem_i1)
        qn1 = _filter(1)
        _start_idx(1, sem_i1, b0 + 3)
        g1 = _start_gather(1, sem_g1, 0)

        _process(0, sem_g0, qn0, g0)
        _process(1, sem_g1, qn1, g1)
        return carry

    lax.fori_loop(0, _NBLK // 2, _pair, 0)

    _wait_idx(0, sem_i0)
    _wait_idx(1, sem_i1)

    pltpu.sync_copy(acc.at[pl.ds(0, _NLOC)], c_hbm.at[pl.ds(lo, _NLOC)])


def _edge_phase(ab, bh, src, dst):
    mesh = plsc.VectorSubcoreMesh(core_axis_name="c", subcore_axis_name="s")
    kern = functools.partial(
        pl.kernel,
        out_type=jax.ShapeDtypeStruct((_NPAD_C, _D), jnp.float32),
        mesh=mesh,
        scratch_types=[
            pltpu.VMEM((_RPW, _D), jnp.float32),
            pltpu.VMEM((_RPW, _D), jnp.float32),
            pltpu.VMEM((2 * _B,), jnp.int32),
            pltpu.VMEM((2 * _B,), jnp.int32),
            pltpu.VMEM((2 * (_B + _K),), jnp.int32),
            pltpu.VMEM((2 * (_B + _K),), jnp.int32),
            pltpu.VMEM((2 * _K, 2 * _D), jnp.float32),
            pltpu.SemaphoreType.DMA,
            pltpu.SemaphoreType.DMA,
            pltpu.SemaphoreType.DMA,
            pltpu.SemaphoreType.DMA,
        ],
        compiler_params=pltpu.CompilerParams(needs_layout_passes=False),
    )(_edge_body)
    return kern(ab, bh, src, dst)


def _apply_body(x_ref, c_ref, w_ref, bap_ref, o_ref):
    h = x_ref[...]
    c = c_ref[...]
    b = (
        jnp.dot(h, w_ref[: _D, :], preferred_element_type=jnp.float32)
        + jnp.dot(c, w_ref[_D :, :], preferred_element_type=jnp.float32)
        + bap_ref[...]
    )
    nrm = jnp.sqrt(jnp.sum(b * b, axis=1, keepdims=True))
    b = b / jnp.maximum(nrm, 1e-12)
    o_ref[...] = jnp.maximum(b, 0.0)


def _apply_phase(x, c_pad, W_apply, b_apply):
    bm = 2000
    grid = (_N // bm,)
    return pl.pallas_call(
        _apply_body,
        grid=grid,
        in_specs=[
            pl.BlockSpec((bm, _D), lambda i: (i, 0)),
            pl.BlockSpec((bm, _D), lambda i: (i, 0)),
            pl.BlockSpec((2 * _D, _D), lambda i: (0, 0)),
            pl.BlockSpec((1, _D), lambda i: (0, 0)),
        ],
        out_specs=pl.BlockSpec((bm, _D), lambda i: (i, 0)),
        out_shape=jax.ShapeDtypeStruct((_N, _D), jnp.float32),
    )(x, c_pad, W_apply, b_apply)


def kernel(x, edge_index, W_A, b_A, W_B, b_B, W_apply, b_apply):
    ab, bh = _mm_phase(
        x, W_A, b_A.reshape(1, _D), W_B, b_B.reshape(1, _D)
    )
    c_pad = _edge_phase(ab, bh, edge_index[0], edge_index[1])
    return _apply_phase(x, c_pad, W_apply, b_apply.reshape(1, _D))


# speedup vs baseline: 6.4612x; 6.0156x over previous
import functools

import jax
import jax.numpy as jnp
from jax import lax
from jax.experimental import pallas as pl
from jax.experimental.pallas import tpu as pltpu
from jax.experimental.pallas import tpu_sc as plsc

_N = 10000
_E = 320000
_D = 128
_NW = 32
_NLOC = 320
_RPW = 328
_NPAD_BH = 10248
_NPAD_C = 10240
_B = 1280
_NBLK = _E // _B
_K = 64
_SS = 8
_G = _D // 16


def _mm_body(x_ref, wa_ref, ba_ref, wb_ref, bb_ref, ab_ref, bh_ref):
    x = x_ref[...]
    ah = jnp.dot(x, wa_ref[...], preferred_element_type=jnp.float32) + ba_ref[...]
    bh = jnp.dot(x, wb_ref[...], preferred_element_type=jnp.float32) + bb_ref[...]
    ab_ref[:, : _D] = jnp.maximum(ah, 0.0)
    ab_ref[:, _D :] = bh
    bh_ref[...] = bh


def _mm_phase(x, W_A, b_A, W_B, b_B):
    bm = 2000
    grid = (_N // bm,)
    return pl.pallas_call(
        _mm_body,
        grid=grid,
        in_specs=[
            pl.BlockSpec((bm, _D), lambda i: (i, 0)),
            pl.BlockSpec((_D, _D), lambda i: (0, 0)),
            pl.BlockSpec((1, _D), lambda i: (0, 0)),
            pl.BlockSpec((_D, _D), lambda i: (0, 0)),
            pl.BlockSpec((1, _D), lambda i: (0, 0)),
        ],
        out_specs=[
            pl.BlockSpec((bm, 2 * _D), lambda i: (i, 0)),
            pl.BlockSpec((bm, _D), lambda i: (i, 0)),
        ],
        out_shape=[
            jax.ShapeDtypeStruct((_N, 2 * _D), jnp.float32),
            jax.ShapeDtypeStruct((_NPAD_BH, _D), jnp.float32),
        ],
    )(x, W_A, b_A, W_B, b_B)


def _edge_body(ab_hbm, bh_hbm, src_hbm, dst_hbm, c_hbm,
               acc, bh_loc, src_blk, dst_blk, q_src, q_loc, rows,
               sem_i0, sem_i1, sem_g0, sem_g1):
    wid = lax.axis_index("s") * 2 + lax.axis_index("c")
    lo = wid * _NLOC

    zero_f = jnp.zeros((16,), jnp.float32)

    def _zero_row(r, carry):
        for g in range(_G):
            acc[r, pl.ds(g * 16, 16)] = zero_f
        return carry

    lax.fori_loop(0, _RPW, _zero_row, 0)

    pltpu.sync_copy(bh_hbm.at[pl.ds(lo, _RPW)], bh_loc)

    zero_i = jnp.zeros((16,), jnp.int32)
    trash_i = jnp.full((16,), _NLOC, jnp.int32)

    _QW = _B + _K

    def _start_idx(par, sem, b):
        off = jnp.minimum(b, _NBLK - 1) * _B
        pltpu.async_copy(
            src_hbm.at[pl.ds(off, _B)], src_blk.at[pl.ds(par * _B, _B)], sem
        )
        pltpu.async_copy(
            dst_hbm.at[pl.ds(off, _B)], dst_blk.at[pl.ds(par * _B, _B)], sem
        )

    def _wait_idx(par, sem):
        pltpu.make_async_copy(
            src_hbm.at[pl.ds(0, _B)], src_blk.at[pl.ds(par * _B, _B)], sem
        ).wait()
        pltpu.make_async_copy(
            dst_hbm.at[pl.ds(0, _B)], dst_blk.at[pl.ds(par * _B, _B)], sem
        ).wait()

    def _filter(par):
        def _filt(i, qn):
            s = src_blk[pl.ds(par * _B + i * 16, 16)]
            d = dst_blk[pl.ds(par * _B + i * 16, 16)]
            locv = d - lo
            mask = (locv >= 0) & (locv < _NLOC)
            csum = plsc.cumsum(mask.astype(jnp.int32))
            pos = csum + (par * _QW + qn - 1)
            plsc.store_scatter(q_src, [pos], s, mask=mask)
            plsc.store_scatter(q_loc, [pos], locv, mask=mask)
            return qn + csum[15]

        qn = lax.fori_loop(0, _B // 16, _filt, jnp.int32(0))
        for t in range(_K // 16):
            q_src[pl.ds(par * _QW + qn + t * 16, 16)] = zero_i
            q_loc[pl.ds(par * _QW + qn + t * 16, 16)] = trash_i
        return qn

    def _nstreams(qn, cbase):
        return jnp.clip((qn - cbase + _SS - 1) // _SS, 0, _K // _SS)

    def _fire_gathers(par, sem, cbase, ns):
        def _fire(i, c):
            pltpu.async_copy(
                ab_hbm.at[q_src.at[pl.ds(par * _QW + cbase + i * _SS, _SS)]],
                rows.at[pl.ds(par * _K + i * _SS, _SS)],
                sem,
            )
            return c

        lax.fori_loop(0, ns, _fire, 0)

    def _drain_gathers(par, sem, ns):
        def _drain(i, c):
            pltpu.make_async_copy(
                ab_hbm.at[q_src.at[pl.ds(par * _QW, _SS)]],
                rows.at[pl.ds(par * _K, _SS)],
                sem,
            ).wait()
            return c

        lax.fori_loop(0, ns, _drain, 0)

    def _grp(par, cbase, g16, carry):
        qbase = par * _QW + cbase + g16 * 16
        locv = q_loc[pl.ds(qbase, 16)]
        for j in range(16):
            e = par * _K + g16 * 16 + j
            locj = locv[j]
            bs = [rows[e, pl.ds(_D + g * 16, 16)] for g in range(_G)]
            bd = [bh_loc[locj, pl.ds(g * 16, 16)] for g in range(_G)]
            ex = [jnp.exp(-(bs[g] + bd[g])) for g in range(_G)]
            den = [1.0 + ex[g] for g in range(_G)]
            sig = [1.0 / den[g] for g in range(_G)]
            ap = [rows[e, pl.ds(g * 16, 16)] for g in range(_G)]
            m = [sig[g] * ap[g] for g in range(_G)]
            cur = [acc[locj, pl.ds(g * 16, 16)] for g in range(_G)]
            for g in range(_G):
                acc[locj, pl.ds(g * 16, 16)] = jnp.maximum(cur[g], m[g])
        return carry

    def _process(par, sem, qn):
        ngrp_total = (qn + 15) // 16
        nchunks = (qn + _K - 1) // _K

        def _chunk(ch, carry):
            cbase = ch * _K
            ns = _nstreams(qn, cbase)

            @pl.when(ch > 0)
            def _():
                _fire_gathers(par, sem, cbase, ns)

            _drain_gathers(par, sem, ns)
            ngrp = jnp.minimum(ngrp_total - ch * (_K // 16), _K // 16)
            lax.fori_loop(0, ngrp, lambda g16, c: _grp(par, cbase, g16, c), 0)
            return carry

        lax.fori_loop(0, nchunks, _chunk, 0)

    _start_idx(0, sem_i0, jnp.int32(0))
    _start_idx(1, sem_i1, jnp.int32(1))

    def _pair(p, carry):
        b0 = 2 * p
        _wait_idx(0, sem_i0)
        qn0 = _filter(0)
        _start_idx(0, sem_i0, b0 + 2)
        _fire_gathers(0, sem_g0, 0, _nstreams(qn0, 0))

        _wait_idx(1, sem_i1)
        qn1 = _filter(1)
        _start_idx(1, sem_i1, b0 + 3)
        _fire_gathers(1, sem_g1, 0, _nstreams(qn1, 0))

        _process(0, sem_g0, qn0)
        _process(1, sem_g1, qn1)
        return carry

    lax.fori_loop(0, _NBLK // 2, _pair, 0)

    _wait_idx(0, sem_i0)
    _wait_idx(1, sem_i1)

    pltpu.sync_copy(acc.at[pl.ds(0, _NLOC)], c_hbm.at[pl.ds(lo, _NLOC)])


def _edge_phase(ab, bh, src, dst):
    mesh = plsc.VectorSubcoreMesh(core_axis_name="c", subcore_axis_name="s")
    kern = functools.partial(
        pl.kernel,
        out_type=jax.ShapeDtypeStruct((_NPAD_C, _D), jnp.float32),
        mesh=mesh,
        scratch_types=[
            pltpu.VMEM((_RPW, _D), jnp.float32),
            pltpu.VMEM((_RPW, _D), jnp.float32),
            pltpu.VMEM((2 * _B,), jnp.int32),
            pltpu.VMEM((2 * _B,), jnp.int32),
            pltpu.VMEM((2 * (_B + _K),), jnp.int32),
            pltpu.VMEM((2 * (_B + _K),), jnp.int32),
            pltpu.VMEM((2 * _K, 2 * _D), jnp.float32),
            pltpu.SemaphoreType.DMA,
            pltpu.SemaphoreType.DMA,
            pltpu.SemaphoreType.DMA,
            pltpu.SemaphoreType.DMA,
        ],
        compiler_params=pltpu.CompilerParams(needs_layout_passes=False),
    )(_edge_body)
    return kern(ab, bh, src, dst)


def _apply_body(x_ref, c_ref, w_ref, bap_ref, o_ref):
    h = x_ref[...]
    c = c_ref[...]
    b = (
        jnp.dot(h, w_ref[: _D, :], preferred_element_type=jnp.float32)
        + jnp.dot(c, w_ref[_D :, :], preferred_element_type=jnp.float32)
        + bap_ref[...]
    )
    nrm = jnp.sqrt(jnp.sum(b * b, axis=1, keepdims=True))
    b = b / jnp.maximum(nrm, 1e-12)
    o_ref[...] = jnp.maximum(b, 0.0)


def _apply_phase(x, c_pad, W_apply, b_apply):
    bm = 2000
    grid = (_N // bm,)
    return pl.pallas_call(
        _apply_body,
        grid=grid,
        in_specs=[
            pl.BlockSpec((bm, _D), lambda i: (i, 0)),
            pl.BlockSpec((bm, _D), lambda i: (i, 0)),
            pl.BlockSpec((2 * _D, _D), lambda i: (0, 0)),
            pl.BlockSpec((1, _D), lambda i: (0, 0)),
        ],
        out_specs=pl.BlockSpec((bm, _D), lambda i: (i, 0)),
        out_shape=jax.ShapeDtypeStruct((_N, _D), jnp.float32),
    )(x, c_pad, W_apply, b_apply)


def kernel(x, edge_index, W_A, b_A, W_B, b_B, W_apply, b_apply):
    ab, bh = _mm_phase(
        x, W_A, b_A.reshape(1, _D), W_B, b_B.reshape(1, _D)
    )
    c_pad = _edge_phase(ab, bh, edge_index[0], edge_index[1])
    return _apply_phase(x, c_pad, W_apply, b_apply.reshape(1, _D))


# speedup vs baseline: 8.4673x; 1.3105x over previous
import functools

import jax
import jax.numpy as jnp
from jax import lax
from jax.experimental import pallas as pl
from jax.experimental.pallas import tpu as pltpu
from jax.experimental.pallas import tpu_sc as plsc

_N = 10000
_E = 320000
_D = 128
_NW = 32
_NLOC = 320
_NPAD_BH = 10248
_NPAD_C = 10240
_B = 2000
_NBLK = _E // _B
_K = 80
_SS = 8
_G = _D // 16


def _mm_body(x_ref, wa_ref, ba_ref, wb_ref, bb_ref, ab_ref, bh_ref):
    x = x_ref[...]
    ah = jnp.dot(x, wa_ref[...], preferred_element_type=jnp.float32) + ba_ref[...]
    bh = jnp.dot(x, wb_ref[...], preferred_element_type=jnp.float32) + bb_ref[...]
    ab_ref[:, : _D] = jnp.maximum(ah, 0.0)
    ab_ref[:, _D :] = bh
    bh_ref[...] = bh


def _mm_phase(x, W_A, b_A, W_B, b_B):
    bm = 2000
    grid = (_N // bm,)
    return pl.pallas_call(
        _mm_body,
        grid=grid,
        in_specs=[
            pl.BlockSpec((bm, _D), lambda i: (i, 0)),
            pl.BlockSpec((_D, _D), lambda i: (0, 0)),
            pl.BlockSpec((1, _D), lambda i: (0, 0)),
            pl.BlockSpec((_D, _D), lambda i: (0, 0)),
            pl.BlockSpec((1, _D), lambda i: (0, 0)),
        ],
        out_specs=[
            pl.BlockSpec((bm, 2 * _D), lambda i: (i, 0)),
            pl.BlockSpec((bm, _D), lambda i: (i, 0)),
        ],
        out_shape=[
            jax.ShapeDtypeStruct((_N, 2 * _D), jnp.float32),
            jax.ShapeDtypeStruct((_NPAD_BH, _D), jnp.float32),
        ],
    )(x, W_A, b_A, W_B, b_B)


def _edge_body(ab_hbm, bh_hbm, src_hbm, dst_hbm, c_hbm,
               acc, src_blk, dst_blk, q_src, q_dst, rows, rows2,
               sem_i0, sem_i1, sem_g0, sem_g1):
    wid = lax.axis_index("s") * 2 + lax.axis_index("c")
    lo = wid * _NLOC

    zero_f = jnp.zeros((16,), jnp.float32)

    def _zero_row(r, carry):
        for g in range(_G):
            acc[r, pl.ds(g * 16, 16)] = zero_f
        return carry

    lax.fori_loop(0, _NLOC + 1, _zero_row, 0)

    zero_i = jnp.zeros((16,), jnp.int32)
    trash_i = jnp.full((16,), lo + _NLOC, jnp.int32)

    _QW = _B + 16

    def _start_idx(par, sem, b):
        off = jnp.minimum(b, _NBLK - 1) * _B
        pltpu.async_copy(
            src_hbm.at[pl.ds(off, _B)], src_blk.at[pl.ds(par * _B, _B)], sem
        )
        pltpu.async_copy(
            dst_hbm.at[pl.ds(off, _B)], dst_blk.at[pl.ds(par * _B, _B)], sem
        )

    def _wait_idx(par, sem):
        pltpu.make_async_copy(
            src_hbm.at[pl.ds(0, _B)], src_blk.at[pl.ds(par * _B, _B)], sem
        ).wait()
        pltpu.make_async_copy(
            dst_hbm.at[pl.ds(0, _B)], dst_blk.at[pl.ds(par * _B, _B)], sem
        ).wait()

    def _filter(par):
        def _filt(i, qn):
            s = src_blk[pl.ds(par * _B + i * 16, 16)]
            d = dst_blk[pl.ds(par * _B + i * 16, 16)]
            locv = d - lo
            mask = (locv >= 0) & (locv < _NLOC)
            csum = plsc.cumsum(mask.astype(jnp.int32))
            pos = csum + (par * _QW + qn - 1)
            plsc.store_scatter(q_src, [pos], s, mask=mask)
            plsc.store_scatter(q_dst, [pos], d, mask=mask)
            return qn + csum[15]

        qn = lax.fori_loop(0, _B // 16, _filt, jnp.int32(0))
        q_src[pl.ds(par * _QW + qn, 16)] = zero_i
        q_dst[pl.ds(par * _QW + qn, 16)] = trash_i
        return qn

    def _nstreams(qn, cbase):
        return jnp.clip((qn - cbase + _SS - 1) // _SS, 0, _K // _SS)

    def _fire_gathers(par, sem, cbase, ns):
        def _fire(i, c):
            pltpu.async_copy(
                ab_hbm.at[q_src.at[pl.ds(par * _QW + cbase + i * _SS, _SS)]],
                rows.at[pl.ds(par * _K + i * _SS, _SS)],
                sem,
            )
            pltpu.async_copy(
                bh_hbm.at[q_dst.at[pl.ds(par * _QW + cbase + i * _SS, _SS)]],
                rows2.at[pl.ds(par * _K + i * _SS, _SS)],
                sem,
            )
            return c

        lax.fori_loop(0, ns, _fire, 0)

    def _drain_gathers(par, sem, ns):
        def _drain(i, c):
            pltpu.make_async_copy(
                ab_hbm.at[q_src.at[pl.ds(par * _QW, _SS)]],
                rows.at[pl.ds(par * _K, _SS)],
                sem,
            ).wait()
            pltpu.make_async_copy(
                bh_hbm.at[q_dst.at[pl.ds(par * _QW, _SS)]],
                rows2.at[pl.ds(par * _K, _SS)],
                sem,
            ).wait()
            return c

        lax.fori_loop(0, ns, _drain, 0)

    def _grp(par, cbase, g16, carry):
        qbase = par * _QW + cbase + g16 * 16
        locv = q_dst[pl.ds(qbase, 16)] - lo
        for j in range(16):
            e = par * _K + g16 * 16 + j
            locj = locv[j]
            bs = [rows[e, pl.ds(_D + g * 16, 16)] for g in range(_G)]
            bd = [rows2[e, pl.ds(g * 16, 16)] for g in range(_G)]
            ex = [jnp.exp(-(bs[g] + bd[g])) for g in range(_G)]
            den = [1.0 + ex[g] for g in range(_G)]
            sig = [1.0 / den[g] for g in range(_G)]
            ap = [rows[e, pl.ds(g * 16, 16)] for g in range(_G)]
            m = [sig[g] * ap[g] for g in range(_G)]
            cur = [acc[locj, pl.ds(g * 16, 16)] for g in range(_G)]
            for g in range(_G):
                acc[locj, pl.ds(g * 16, 16)] = jnp.maximum(cur[g], m[g])
        return carry

    def _process(par, sem, qn):
        ngrp_total = (qn + 15) // 16
        nchunks = (qn + _K - 1) // _K

        def _chunk(ch, carry):
            cbase = ch * _K
            ns = _nstreams(qn, cbase)

            @pl.when(ch > 0)
            def _():
                _fire_gathers(par, sem, cbase, ns)

            _drain_gathers(par, sem, ns)
            ngrp = jnp.minimum(ngrp_total - ch * (_K // 16), _K // 16)
            lax.fori_loop(0, ngrp, lambda g16, c: _grp(par, cbase, g16, c), 0)
            return carry

        lax.fori_loop(0, nchunks, _chunk, 0)

    _start_idx(0, sem_i0, jnp.int32(0))
    _start_idx(1, sem_i1, jnp.int32(1))

    def _pair(p, carry):
        b0 = 2 * p
        _wait_idx(0, sem_i0)
        qn0 = _filter(0)
        _start_idx(0, sem_i0, b0 + 2)
        _fire_gathers(0, sem_g0, 0, _nstreams(qn0, 0))

        _wait_idx(1, sem_i1)
        qn1 = _filter(1)
        _start_idx(1, sem_i1, b0 + 3)
        _fire_gathers(1, sem_g1, 0, _nstreams(qn1, 0))

        _process(0, sem_g0, qn0)
        _process(1, sem_g1, qn1)
        return carry

    lax.fori_loop(0, _NBLK // 2, _pair, 0)

    _wait_idx(0, sem_i0)
    _wait_idx(1, sem_i1)

    pltpu.sync_copy(acc.at[pl.ds(0, _NLOC)], c_hbm.at[pl.ds(lo, _NLOC)])


def _edge_phase(ab, bh, src, dst):
    mesh = plsc.VectorSubcoreMesh(core_axis_name="c", subcore_axis_name="s")
    kern = functools.partial(
        pl.kernel,
        out_type=jax.ShapeDtypeStruct((_NPAD_C, _D), jnp.float32),
        mesh=mesh,
        scratch_types=[
            pltpu.VMEM((_NLOC + 1, _D), jnp.float32),
            pltpu.VMEM((2 * _B,), jnp.int32),
            pltpu.VMEM((2 * _B,), jnp.int32),
            pltpu.VMEM((2 * (_B + 16),), jnp.int32),
            pltpu.VMEM((2 * (_B + 16),), jnp.int32),
            pltpu.VMEM((2 * _K, 2 * _D), jnp.float32),
            pltpu.VMEM((2 * _K, _D), jnp.float32),
            pltpu.SemaphoreType.DMA,
            pltpu.SemaphoreType.DMA,
            pltpu.SemaphoreType.DMA,
            pltpu.SemaphoreType.DMA,
        ],
        compiler_params=pltpu.CompilerParams(needs_layout_passes=False),
    )(_edge_body)
    return kern(ab, bh, src, dst)


def _apply_body(x_ref, c_ref, w_ref, bap_ref, o_ref):
    h = x_ref[...]
    c = c_ref[...]
    b = (
        jnp.dot(h, w_ref[: _D, :], preferred_element_type=jnp.float32)
        + jnp.dot(c, w_ref[_D :, :], preferred_element_type=jnp.float32)
        + bap_ref[...]
    )
    nrm = jnp.sqrt(jnp.sum(b * b, axis=1, keepdims=True))
    b = b / jnp.maximum(nrm, 1e-12)
    o_ref[...] = jnp.maximum(b, 0.0)


def _apply_phase(x, c_pad, W_apply, b_apply):
    bm = 2000
    grid = (_N // bm,)
    return pl.pallas_call(
        _apply_body,
        grid=grid,
        in_specs=[
            pl.BlockSpec((bm, _D), lambda i: (i, 0)),
            pl.BlockSpec((bm, _D), lambda i: (i, 0)),
            pl.BlockSpec((2 * _D, _D), lambda i: (0, 0)),
            pl.BlockSpec((1, _D), lambda i: (0, 0)),
        ],
        out_specs=pl.BlockSpec((bm, _D), lambda i: (i, 0)),
        out_shape=jax.ShapeDtypeStruct((_N, _D), jnp.float32),
    )(x, c_pad, W_apply, b_apply)


def kernel(x, edge_index, W_A, b_A, W_B, b_B, W_apply, b_apply):
    ab, bh = _mm_phase(
        x, W_A, b_A.reshape(1, _D), W_B, b_B.reshape(1, _D)
    )
    c_pad = _edge_phase(ab, bh, edge_index[0], edge_index[1])
    return _apply_phase(x, c_pad, W_apply, b_apply.reshape(1, _D))
